# baseline (speedup 1.0000x reference)
import functools

import jax
import jax.numpy as jnp
from jax import lax
from jax.experimental import pallas as pl
from jax.experimental.pallas import tpu as pltpu
from jax.experimental.pallas import tpu_sc as plsc

N = 10000
E = 320000
IN_DIM = 128
EMB = 64
EDGE = 16
EPS = 1e-5

NC = 2
NS = 16
NW = NC * NS

KG = 64
NBPW_G = 160
GSLOT = 8

KB = 128
NBPW = 80
NSLOT = 4

E_PAD = NW * NBPW * KB
N_PAD = 10240
ROWS_PER_TILE = N_PAD // NS
PAD_SINK = N_PAD - 8

NBLK_TC = 125
BE = E // NBLK_TC
RC = 1

_mesh = plsc.VectorSubcoreMesh(core_axis_name="c", subcore_axis_name="s")


def _sc_gather_body(hd, hs, dst3, src3, g_out,
                    idxd, idxs, bufd, bufs, sbuf, gsemd, gsems, ssem):
    c = lax.axis_index("c")
    s = lax.axis_index("s")
    w = s * NC + c
    pltpu.sync_copy(dst3.at[w], idxd)
    pltpu.sync_copy(src3.at[w], idxs)
    wb = w * NBPW_G

    cp = lax.rem(w, RC)

    def fire(b, k):
        pltpu.async_copy(hd.at[cp].at[idxd.at[b]], bufd.at[k], gsemd.at[k])
        pltpu.async_copy(hs.at[cp].at[idxs.at[b]], bufs.at[k], gsems.at[k])

    def wait_gather(b, k):
        pltpu.make_async_copy(hd.at[cp].at[idxd.at[b]], bufd.at[k],
                              gsemd.at[k]).wait()
        pltpu.make_async_copy(hs.at[cp].at[idxs.at[b]], bufs.at[k],
                              gsems.at[k]).wait()

    def store(b, sk):
        return pltpu.async_copy(
            sbuf.at[sk], g_out.at[pl.ds((wb + b) * KG, KG)], ssem.at[sk])

    def wait_store(b, sk):
        pltpu.make_async_copy(
            sbuf.at[sk], g_out.at[pl.ds((wb + b) * KG, KG)], ssem.at[sk]).wait()

    for k in range(GSLOT):
        fire(k, k)

    NIT = NBPW_G // GSLOT

    def it_body(it, carry):
        for k in range(GSLOT):
            b = GSLOT * it + k
            sk = k % 4
            wait_gather(b, k)

            @pl.when(b >= 4)
            def _():
                wait_store(b - 4, sk)

            def row(r):
                for q in range(4):
                    sl = pl.ds(q * 16, 16)
                    sbuf[sk, r, sl] = bufd[k, r, sl] + bufs[k, r, sl]

            plsc.parallel_loop(0, KG, 1, unroll=4)(row)
            store(b, sk)

            @pl.when(it < NIT - 1)
            def _():
                fire(b + GSLOT, k)
        return carry

    lax.fori_loop(0, NIT, it_body, 0)
    for k in range(4):
        b = NBPW_G - 4 + k
        wait_store(b, b % 4)


_sc_gather = functools.partial(
    pl.kernel,
    _sc_gather_body,
    mesh=_mesh,
    compiler_params=pltpu.CompilerParams(use_tc_tiling_on_sc=False),
    out_type=jax.ShapeDtypeStruct((E_PAD, EMB), jnp.float32),
    scratch_types=[
        pltpu.VMEM((NBPW_G, KG), jnp.int32),
        pltpu.VMEM((NBPW_G, KG), jnp.int32),
        pltpu.VMEM((GSLOT, KG, EMB), jnp.float32),
        pltpu.VMEM((GSLOT, KG, EMB), jnp.float32),
        pltpu.VMEM((4, KG, EMB), jnp.float32),
        pltpu.SemaphoreType.DMA((GSLOT,)),
        pltpu.SemaphoreType.DMA((GSLOT,)),
        pltpu.SemaphoreType.DMA((4,)),
    ],
)()


def _sc_scatter_body(y2, dsts3, ss2, out, idx, bufp, sbuf, zbuf, ssv, aggr,
                     lsem, scsem):
    c = lax.axis_index("c")
    s = lax.axis_index("s")
    w = s * NC + c
    pltpu.sync_copy(dsts3.at[w], idx)
    pltpu.sync_copy(ss2, ssv)
    wb = w * NBPW

    def load(b, k):
        pltpu.async_copy(y2.at[pl.ds((wb + b) * KB, KB)], bufp.at[k],
                         lsem.at[k])

    def wait_load(b, k):
        pltpu.make_async_copy(y2.at[pl.ds((wb + b) * KB, KB)], bufp.at[k],
                              lsem.at[k]).wait()

    def scat(b, sk):
        pltpu.async_copy(sbuf.at[sk], aggr.at[idx.at[b]], scsem.at[sk],
                         add=True)

    def wait_scat(b, sk):
        pltpu.make_async_copy(sbuf.at[sk], aggr.at[idx.at[b]],
                              scsem.at[sk]).wait()

    for k in range(NSLOT):
        load(k, k)

    zero = jnp.zeros((16,), jnp.float32)

    def zrow(r, cr):
        for q in range(4):
            zbuf[r, pl.ds(q * 16, 16)] = zero
        return cr

    lax.fori_loop(0, KB, zrow, 0)
    for k in range(ROWS_PER_TILE // KB):
        pltpu.sync_copy(zbuf, aggr.at[pl.ds(s * ROWS_PER_TILE + k * KB, KB)])
    plsc.subcore_barrier()

    scv = [ssv[0, pl.ds(q * 16, 16)] for q in range(4)]
    shv = [ssv[1, pl.ds(q * 16, 16)] for q in range(4)]

    NIT = NBPW // NSLOT

    def it_body(it, carry):
        for k in range(NSLOT):
            b = NSLOT * it + k
            sk = k % 2
            wait_load(b, k)

            @pl.when(b >= 2)
            def _():
                wait_scat(b - 2, sk)

            def row(r):
                for q in range(4):
                    sl = pl.ds(q * 16, 16)
                    sbuf[sk, r, sl] = jnp.maximum(
                        bufp[k, r, sl] * scv[q] + shv[q], 0.0)

            plsc.parallel_loop(0, KB, 1, unroll=4)(row)
            scat(b, sk)

            @pl.when(it < NIT - 1)
            def _():
                load(b + NSLOT, k)
        return carry

    lax.fori_loop(0, NIT, it_body, 0)
    for k in range(2):
        b = NBPW - 2 + k
        wait_scat(b, b % 2)
    plsc.subcore_barrier()

    last_rows = N - (NS - 1) * ROWS_PER_TILE

    @pl.when(s < NS - 1)
    def _():
        pltpu.sync_copy(aggr.at[pl.ds(s * ROWS_PER_TILE, ROWS_PER_TILE)],
                        out.at[c, pl.ds(s * ROWS_PER_TILE, ROWS_PER_TILE)])

    @pl.when(s == NS - 1)
    def _():
        pltpu.sync_copy(aggr.at[pl.ds(s * ROWS_PER_TILE, last_rows)],
                        out.at[c, pl.ds(s * ROWS_PER_TILE, last_rows)])


_sc_scatter = functools.partial(
    pl.kernel,
    _sc_scatter_body,
    mesh=_mesh,
    compiler_params=pltpu.CompilerParams(use_tc_tiling_on_sc=False),
    out_type=jax.ShapeDtypeStruct((NC, N, EMB), jnp.float32),
    scratch_types=[
        pltpu.VMEM((NBPW, KB), jnp.int32),
        pltpu.VMEM((NSLOT, KB, EMB), jnp.float32),
        pltpu.VMEM((2, KB, EMB), jnp.float32),
        pltpu.VMEM((KB, EMB), jnp.float32),
        pltpu.VMEM((2, EMB), jnp.float32),
        pltpu.VMEM_SHARED((N_PAD, EMB), jnp.float32),
        pltpu.SemaphoreType.DMA((NSLOT,)),
        pltpu.SemaphoreType.DMA((2,)),
    ],
)()


def _init_body(x_ref, wi_ref, bi_ref, h_ref):
    h = jnp.dot(x_ref[...], wi_ref[...], preferred_element_type=jnp.float32)
    h_ref[...] = h + bi_ref[...]


def _tc_init(x, wi, bi):
    return pl.pallas_call(
        _init_body,
        out_shape=jax.ShapeDtypeStruct((N, EMB), jnp.float32),
    )(x, wi, bi)


def _tables_body(h_ref, wd_ref, ws_ref, hd_ref, hs_ref):
    h = h_ref[...]
    hd_ref[0] = jnp.dot(h, wd_ref[...], preferred_element_type=jnp.float32)
    hs_ref[0] = jnp.dot(h, ws_ref[...], preferred_element_type=jnp.float32)


def _tc_tables(h, wd, ws):
    return pl.pallas_call(
        _tables_body,
        grid=(RC,),
        in_specs=[
            pl.BlockSpec((N, EMB), lambda i: (0, 0)),
            pl.BlockSpec((EMB, EMB), lambda i: (0, 0)),
            pl.BlockSpec((EMB, EMB), lambda i: (0, 0)),
        ],
        out_specs=[
            pl.BlockSpec((1, N, EMB), lambda i: (i, 0, 0)),
            pl.BlockSpec((1, N, EMB), lambda i: (i, 0, 0)),
        ],
        out_shape=[jax.ShapeDtypeStruct((RC, N, EMB), jnp.float32)] * 2,
    )(h, wd, ws)


def _msg1_body(g_ref, ea_ref, wc_ref, b1_ref, g1_ref, be1_ref,
               ss_ref, acc_ref):
    i = pl.program_id(0)
    y = g_ref[...] + jnp.dot(ea_ref[...], wc_ref[...],
                             preferred_element_type=jnp.float32) + b1_ref[...]
    st = jnp.stack([jnp.sum(y, axis=0), jnp.sum(y * y, axis=0)])
    tot = jnp.where(i == 0, st, acc_ref[0:2, :] + st)
    acc_ref[0:2, :] = tot
    mu = tot[0] / E
    var = tot[1] / E - mu * mu
    sc = g1_ref[0] * lax.rsqrt(var + EPS)
    sh = be1_ref[0] - mu * sc
    ss_ref[...] = jnp.stack([sc, sh])


def _tc_msg1(g, ea, wc, b1, g1, be1):
    return pl.pallas_call(
        _msg1_body,
        grid=(NBLK_TC,),
        compiler_params=pltpu.CompilerParams(
            dimension_semantics=("arbitrary",)),
        in_specs=[
            pl.BlockSpec((BE, EMB), lambda i: (i, 0)),
            pl.BlockSpec((BE, EDGE), lambda i: (i, 0)),
            pl.BlockSpec((EDGE, EMB), lambda i: (0, 0)),
            pl.BlockSpec((1, EMB), lambda i: (0, 0)),
            pl.BlockSpec((1, EMB), lambda i: (0, 0)),
            pl.BlockSpec((1, EMB), lambda i: (0, 0)),
        ],
        out_specs=pl.BlockSpec((2, EMB), lambda i: (0, 0)),
        out_shape=jax.ShapeDtypeStruct((2, EMB), jnp.float32),
        scratch_shapes=[pltpu.VMEM((8, EMB), jnp.float32)],
    )(g, ea, wc, b1, g1, be1)


def _msg2_body(g_ref, ea_ref, wc_ref, b1_ref, ss1_ref, w2_ref, b2_ref,
               g2_ref, be2_ref, y2_ref, ss_ref, acc_ref):
    i = pl.program_id(0)
    y1 = g_ref[...] + jnp.dot(ea_ref[...], wc_ref[...],
                              preferred_element_type=jnp.float32) + b1_ref[...]
    z = jnp.maximum(y1 * ss1_ref[0] + ss1_ref[1], 0.0)
    y = jnp.dot(z, w2_ref[...], preferred_element_type=jnp.float32) + b2_ref[...]
    y2_ref[...] = y
    st = jnp.stack([jnp.sum(y, axis=0), jnp.sum(y * y, axis=0)])
    tot = jnp.where(i == 0, st, acc_ref[0:2, :] + st)
    acc_ref[0:2, :] = tot

    @pl.when(i == NBLK_TC - 1)
    def _():
        mu = tot[0] / E
        var = tot[1] / E - mu * mu
        sc = g2_ref[0] * lax.rsqrt(var + EPS)
        sh = be2_ref[0] - mu * sc
        ss_ref[...] = jnp.stack([sc, sh])


def _tc_msg2(g, ea, wc, b1, ss1, w2, b2, g2, be2):
    return pl.pallas_call(
        _msg2_body,
        grid=(NBLK_TC,),
        compiler_params=pltpu.CompilerParams(
            dimension_semantics=("arbitrary",)),
        in_specs=[
            pl.BlockSpec((BE, EMB), lambda i: (i, 0)),
            pl.BlockSpec((BE, EDGE), lambda i: (i, 0)),
            pl.BlockSpec((EDGE, EMB), lambda i: (0, 0)),
            pl.BlockSpec((1, EMB), lambda i: (0, 0)),
            pl.BlockSpec((2, EMB), lambda i: (0, 0)),
            pl.BlockSpec((EMB, EMB), lambda i: (0, 0)),
            pl.BlockSpec((1, EMB), lambda i: (0, 0)),
            pl.BlockSpec((1, EMB), lambda i: (0, 0)),
            pl.BlockSpec((1, EMB), lambda i: (0, 0)),
        ],
        out_specs=[
            pl.BlockSpec((BE, EMB), lambda i: (i, 0)),
            pl.BlockSpec((2, EMB), lambda i: (0, 0)),
        ],
        out_shape=[
            jax.ShapeDtypeStruct((E_PAD, EMB), jnp.float32),
            jax.ShapeDtypeStruct((2, EMB), jnp.float32),
        ],
        scratch_shapes=[pltpu.VMEM((8, EMB), jnp.float32)],
    )(g, ea, wc, b1, ss1, w2, b2, g2, be2)


def _bn_tc(y, g, be):
    mu = jnp.mean(y, axis=0)
    var = jnp.mean((y - mu) ** 2, axis=0)
    return g * (y - mu) * lax.rsqrt(var + EPS) + be


def _node_mid_body(h_ref, ap_ref, wuh_ref, wua_ref, bu1_ref, gu1_ref, beu1_ref,
                   wu2_ref, bu2_ref, gu2_ref, beu2_ref, h_out):
    h = h_ref[...]
    aggr = ap_ref[0] + ap_ref[1]
    y = (jnp.dot(h, wuh_ref[...], preferred_element_type=jnp.float32)
         + jnp.dot(aggr, wua_ref[...], preferred_element_type=jnp.float32)
         + bu1_ref[...])
    z = jnp.maximum(_bn_tc(y, gu1_ref[...], beu1_ref[...]), 0.0)
    y2 = jnp.dot(z, wu2_ref[...], preferred_element_type=jnp.float32) + bu2_ref[...]
    u = jnp.maximum(_bn_tc(y2, gu2_ref[...], beu2_ref[...]), 0.0)
    h_out[...] = h + u


def _tc_node_mid(h, ap, wuh, wua, bu1, gu1, beu1, wu2, bu2, gu2, beu2):
    return pl.pallas_call(
        _node_mid_body,
        out_shape=jax.ShapeDtypeStruct((N, EMB), jnp.float32),
    )(h, ap, wuh, wua, bu1, gu1, beu1, wu2, bu2, gu2, beu2)


def _node_last_body(h_ref, ap_ref, wuh_ref, wua_ref, bu1_ref, gu1_ref, beu1_ref,
                    wu2_ref, bu2_ref, gu2_ref, beu2_ref, wp_ref, bp_ref,
                    out_ref):
    h = h_ref[...]
    aggr = ap_ref[0] + ap_ref[1]
    y = (jnp.dot(h, wuh_ref[...], preferred_element_type=jnp.float32)
         + jnp.dot(aggr, wua_ref[...], preferred_element_type=jnp.float32)
         + bu1_ref[...])
    z = jnp.maximum(_bn_tc(y, gu1_ref[...], beu1_ref[...]), 0.0)
    y2 = jnp.dot(z, wu2_ref[...], preferred_element_type=jnp.float32) + bu2_ref[...]
    u = jnp.maximum(_bn_tc(y2, gu2_ref[...], beu2_ref[...]), 0.0)
    hn = h + u
    out_ref[...] = jnp.sum(hn * wp_ref[...], axis=1, keepdims=True) + bp_ref[...]


def _tc_node_last(h, ap, wuh, wua, bu1, gu1, beu1, wu2, bu2, gu2, beu2, wp, bp):
    return pl.pallas_call(
        _node_last_body,
        out_shape=jax.ShapeDtypeStruct((N, 1), jnp.float32),
    )(h, ap, wuh, wua, bu1, gu1, beu1, wu2, bu2, gu2, beu2, wp, bp)


def kernel(x, edge_index, edge_attr, params):
    src = edge_index[0]
    dst = edge_index[1]
    pad = E_PAD - E
    dst3 = jnp.pad(dst, (0, pad)).reshape(NW, NBPW_G, KG)
    src3 = jnp.pad(src, (0, pad)).reshape(NW, NBPW_G, KG)
    dsts3 = jnp.pad(dst, (0, pad), constant_values=PAD_SINK).reshape(NW, NBPW, KB)

    row = lambda v: v.reshape(1, EMB)
    lp = params['layers']
    wi = params['lin_in']['W']
    bi = row(params['lin_in']['b'])

    wd = [l['msg']['W1'][:EMB] for l in lp]
    ws = [l['msg']['W1'][EMB:2 * EMB] for l in lp]
    wc = [l['msg']['W1'][2 * EMB:] for l in lp]

    h = _tc_init(x, wi, bi)
    hd, hs = _tc_tables(h, wd[0], ws[0])

    out = None
    for l in range(4):
        m = lp[l]['msg']
        u = lp[l]['upd']
        g = _sc_gather(hd, hs, dst3, src3)
        ss1 = _tc_msg1(g, edge_attr, wc[l], row(m['b1']),
                       row(m['g1']), row(m['be1']))
        y2, ss2 = _tc_msg2(g, edge_attr, wc[l], row(m['b1']), ss1,
                           m['W2'], row(m['b2']),
                           row(m['g2']), row(m['be2']))
        ap = _sc_scatter(y2, dsts3, ss2)
        uargs = (u['W1'][:EMB], u['W1'][EMB:], row(u['b1']),
                 row(u['g1']), row(u['be1']),
                 u['W2'], row(u['b2']), row(u['g2']), row(u['be2']))
        if l < 3:
            h = _tc_node_mid(h, ap, *uargs)
            hd, hs = _tc_tables(h, wd[l + 1], ws[l + 1])
        else:
            wp = params['lin_pred']['W'].reshape(1, EMB)
            bp = params['lin_pred']['b'].reshape(1, 1)
            out = _tc_node_last(h, ap, *uargs, wp, bp)
    return out.reshape(-1)


# speedup vs baseline: 1.0362x; 1.0362x over previous
import functools

import jax
import jax.numpy as jnp
from jax import lax
from jax.experimental import pallas as pl
from jax.experimental.pallas import tpu as pltpu
from jax.experimental.pallas import tpu_sc as plsc

N = 10000
E = 320000
IN_DIM = 128
EMB = 64
EDGE = 16
EPS = 1e-5

NC = 2
NS = 16
NW = NC * NS

KG = 64
NBPW_G = 160
GSLOT = 8

KB = 128
NBPW = 80
NSLOT = 4

E_PAD = NW * NBPW * KB
N_PAD = 10240
ROWS_PER_TILE = N_PAD // NS
PAD_SINK = N_PAD - 8

NBLK_TC = 125
BE = E // NBLK_TC
RC = 1

_mesh = plsc.VectorSubcoreMesh(core_axis_name="c", subcore_axis_name="s")


def _sc_gather_body(hd, hs, dst3, src3, g_out,
                    idxd, idxs, bufd, bufs, sbuf, gsemd, gsems, ssem):
    c = lax.axis_index("c")
    s = lax.axis_index("s")
    w = s * NC + c
    pltpu.sync_copy(dst3.at[w], idxd)
    pltpu.sync_copy(src3.at[w], idxs)
    wb = w * NBPW_G

    cp = lax.rem(w, RC)

    def fire(b, k):
        pltpu.async_copy(hd.at[cp].at[idxd.at[b]], bufd.at[k], gsemd.at[k])
        pltpu.async_copy(hs.at[cp].at[idxs.at[b]], bufs.at[k], gsems.at[k])

    def wait_gather(b, k):
        pltpu.make_async_copy(hd.at[cp].at[idxd.at[b]], bufd.at[k],
                              gsemd.at[k]).wait()
        pltpu.make_async_copy(hs.at[cp].at[idxs.at[b]], bufs.at[k],
                              gsems.at[k]).wait()

    def store(b, sk):
        return pltpu.async_copy(
            sbuf.at[sk], g_out.at[pl.ds((wb + b) * KG, KG)], ssem.at[sk])

    def wait_store(b, sk):
        pltpu.make_async_copy(
            sbuf.at[sk], g_out.at[pl.ds((wb + b) * KG, KG)], ssem.at[sk]).wait()

    for k in range(GSLOT):
        fire(k, k)

    NIT = NBPW_G // GSLOT

    def it_body(it, carry):
        for k in range(GSLOT):
            b = GSLOT * it + k
            sk = k % 4
            wait_gather(b, k)

            @pl.when(b >= 4)
            def _():
                wait_store(b - 4, sk)

            def row(r):
                for q in range(4):
                    sl = pl.ds(q * 16, 16)
                    sbuf[sk, r, sl] = bufd[k, r, sl] + bufs[k, r, sl]

            plsc.parallel_loop(0, KG, 1, unroll=4)(row)
            store(b, sk)

            @pl.when(it < NIT - 1)
            def _():
                fire(b + GSLOT, k)
        return carry

    lax.fori_loop(0, NIT, it_body, 0)
    for k in range(4):
        b = NBPW_G - 4 + k
        wait_store(b, b % 4)


_sc_gather = functools.partial(
    pl.kernel,
    _sc_gather_body,
    mesh=_mesh,
    compiler_params=pltpu.CompilerParams(use_tc_tiling_on_sc=False),
    out_type=jax.ShapeDtypeStruct((E_PAD, EMB), jnp.float32),
    scratch_types=[
        pltpu.VMEM((NBPW_G, KG), jnp.int32),
        pltpu.VMEM((NBPW_G, KG), jnp.int32),
        pltpu.VMEM((GSLOT, KG, EMB), jnp.float32),
        pltpu.VMEM((GSLOT, KG, EMB), jnp.float32),
        pltpu.VMEM((4, KG, EMB), jnp.float32),
        pltpu.SemaphoreType.DMA((GSLOT,)),
        pltpu.SemaphoreType.DMA((GSLOT,)),
        pltpu.SemaphoreType.DMA((4,)),
    ],
)()


def _sc_scatter_body(y2, dsts3, ss2, out, idx, bufp, sbuf, zbuf, ssv, aggr,
                     lsem, scsem):
    c = lax.axis_index("c")
    s = lax.axis_index("s")
    w = s * NC + c
    pltpu.sync_copy(dsts3.at[w], idx)
    pltpu.sync_copy(ss2, ssv)
    wb = w * NBPW

    def load(b, k):
        pltpu.async_copy(y2.at[pl.ds((wb + b) * KB, KB)], bufp.at[k],
                         lsem.at[k])

    def wait_load(b, k):
        pltpu.make_async_copy(y2.at[pl.ds((wb + b) * KB, KB)], bufp.at[k],
                              lsem.at[k]).wait()

    def scat(b, sk):
        pltpu.async_copy(sbuf.at[sk], aggr.at[idx.at[b]], scsem.at[sk],
                         add=True)

    def wait_scat(b, sk):
        pltpu.make_async_copy(sbuf.at[sk], aggr.at[idx.at[b]],
                              scsem.at[sk]).wait()

    for k in range(NSLOT):
        load(k, k)

    zero = jnp.zeros((16,), jnp.float32)

    def zrow(r, cr):
        for q in range(4):
            zbuf[r, pl.ds(q * 16, 16)] = zero
        return cr

    lax.fori_loop(0, KB, zrow, 0)
    for k in range(ROWS_PER_TILE // KB):
        pltpu.sync_copy(zbuf, aggr.at[pl.ds(s * ROWS_PER_TILE + k * KB, KB)])
    plsc.subcore_barrier()

    scv = [ssv[0, pl.ds(q * 16, 16)] for q in range(4)]
    shv = [ssv[1, pl.ds(q * 16, 16)] for q in range(4)]

    NIT = NBPW // NSLOT

    def it_body(it, carry):
        for k in range(NSLOT):
            b = NSLOT * it + k
            sk = k % 2
            wait_load(b, k)

            @pl.when(b >= 2)
            def _():
                wait_scat(b - 2, sk)

            def row(r):
                for q in range(4):
                    sl = pl.ds(q * 16, 16)
                    sbuf[sk, r, sl] = jnp.maximum(
                        bufp[k, r, sl] * scv[q] + shv[q], 0.0)

            plsc.parallel_loop(0, KB, 1, unroll=4)(row)
            scat(b, sk)

            @pl.when(it < NIT - 1)
            def _():
                load(b + NSLOT, k)
        return carry

    lax.fori_loop(0, NIT, it_body, 0)
    for k in range(2):
        b = NBPW - 2 + k
        wait_scat(b, b % 2)
    plsc.subcore_barrier()

    last_rows = N - (NS - 1) * ROWS_PER_TILE

    @pl.when(s < NS - 1)
    def _():
        pltpu.sync_copy(aggr.at[pl.ds(s * ROWS_PER_TILE, ROWS_PER_TILE)],
                        out.at[c, pl.ds(s * ROWS_PER_TILE, ROWS_PER_TILE)])

    @pl.when(s == NS - 1)
    def _():
        pltpu.sync_copy(aggr.at[pl.ds(s * ROWS_PER_TILE, last_rows)],
                        out.at[c, pl.ds(s * ROWS_PER_TILE, last_rows)])


_sc_scatter = functools.partial(
    pl.kernel,
    _sc_scatter_body,
    mesh=_mesh,
    compiler_params=pltpu.CompilerParams(use_tc_tiling_on_sc=False),
    out_type=jax.ShapeDtypeStruct((NC, N, EMB), jnp.float32),
    scratch_types=[
        pltpu.VMEM((NBPW, KB), jnp.int32),
        pltpu.VMEM((NSLOT, KB, EMB), jnp.float32),
        pltpu.VMEM((2, KB, EMB), jnp.float32),
        pltpu.VMEM((KB, EMB), jnp.float32),
        pltpu.VMEM((2, EMB), jnp.float32),
        pltpu.VMEM_SHARED((N_PAD, EMB), jnp.float32),
        pltpu.SemaphoreType.DMA((NSLOT,)),
        pltpu.SemaphoreType.DMA((2,)),
    ],
)()


def _init_body(x_ref, wi_ref, bi_ref, h_ref):
    h = jnp.dot(x_ref[...], wi_ref[...], preferred_element_type=jnp.float32)
    h_ref[...] = h + bi_ref[...]


def _tc_init(x, wi, bi):
    return pl.pallas_call(
        _init_body,
        out_shape=jax.ShapeDtypeStruct((N, EMB), jnp.float32),
    )(x, wi, bi)


def _tables_body(h_ref, wd_ref, ws_ref, hd_ref, hs_ref):
    h = h_ref[...]
    hd_ref[0] = jnp.dot(h, wd_ref[...], preferred_element_type=jnp.float32)
    hs_ref[0] = jnp.dot(h, ws_ref[...], preferred_element_type=jnp.float32)


def _tc_tables(h, wd, ws):
    return pl.pallas_call(
        _tables_body,
        grid=(RC,),
        in_specs=[
            pl.BlockSpec((N, EMB), lambda i: (0, 0)),
            pl.BlockSpec((EMB, EMB), lambda i: (0, 0)),
            pl.BlockSpec((EMB, EMB), lambda i: (0, 0)),
        ],
        out_specs=[
            pl.BlockSpec((1, N, EMB), lambda i: (i, 0, 0)),
            pl.BlockSpec((1, N, EMB), lambda i: (i, 0, 0)),
        ],
        out_shape=[jax.ShapeDtypeStruct((RC, N, EMB), jnp.float32)] * 2,
    )(h, wd, ws)


def _msg1_body(g_ref, ea_ref, wc_ref, b1_ref, g1_ref, be1_ref,
               y1_ref, ss_ref, acc_ref):
    i = pl.program_id(0)
    y = g_ref[...] + jnp.dot(ea_ref[...], wc_ref[...],
                             preferred_element_type=jnp.float32) + b1_ref[...]
    y1_ref[...] = y
    st = jnp.stack([jnp.sum(y, axis=0), jnp.sum(y * y, axis=0)])
    tot = jnp.where(i == 0, st, acc_ref[0:2, :] + st)
    acc_ref[0:2, :] = tot
    mu = tot[0] / E
    var = tot[1] / E - mu * mu
    sc = g1_ref[0] * lax.rsqrt(var + EPS)
    sh = be1_ref[0] - mu * sc
    ss_ref[...] = jnp.stack([sc, sh])


def _tc_msg1(g, ea, wc, b1, g1, be1):
    return pl.pallas_call(
        _msg1_body,
        grid=(NBLK_TC,),
        compiler_params=pltpu.CompilerParams(
            dimension_semantics=("arbitrary",)),
        in_specs=[
            pl.BlockSpec((BE, EMB), lambda i: (i, 0)),
            pl.BlockSpec((BE, EDGE), lambda i: (i, 0)),
            pl.BlockSpec((EDGE, EMB), lambda i: (0, 0)),
            pl.BlockSpec((1, EMB), lambda i: (0, 0)),
            pl.BlockSpec((1, EMB), lambda i: (0, 0)),
            pl.BlockSpec((1, EMB), lambda i: (0, 0)),
        ],
        out_specs=[
            pl.BlockSpec((BE, EMB), lambda i: (i, 0)),
            pl.BlockSpec((2, EMB), lambda i: (0, 0)),
        ],
        out_shape=[
            jax.ShapeDtypeStruct((E_PAD, EMB), jnp.float32),
            jax.ShapeDtypeStruct((2, EMB), jnp.float32),
        ],
        scratch_shapes=[pltpu.VMEM((8, EMB), jnp.float32)],
    )(g, ea, wc, b1, g1, be1)


def _msg2_body(y1_ref, ss1_ref, w2_ref, b2_ref, g2_ref, be2_ref,
               y2_ref, ss_ref, acc_ref):
    i = pl.program_id(0)
    z = jnp.maximum(y1_ref[...] * ss1_ref[0] + ss1_ref[1], 0.0)
    y = jnp.dot(z, w2_ref[...], preferred_element_type=jnp.float32) + b2_ref[...]
    y2_ref[...] = y
    st = jnp.stack([jnp.sum(y, axis=0), jnp.sum(y * y, axis=0)])
    tot = jnp.where(i == 0, st, acc_ref[0:2, :] + st)
    acc_ref[0:2, :] = tot

    @pl.when(i == NBLK_TC - 1)
    def _():
        mu = tot[0] / E
        var = tot[1] / E - mu * mu
        sc = g2_ref[0] * lax.rsqrt(var + EPS)
        sh = be2_ref[0] - mu * sc
        ss_ref[...] = jnp.stack([sc, sh])


def _tc_msg2(y1, ss1, w2, b2, g2, be2):
    return pl.pallas_call(
        _msg2_body,
        grid=(NBLK_TC,),
        compiler_params=pltpu.CompilerParams(
            dimension_semantics=("arbitrary",)),
        in_specs=[
            pl.BlockSpec((BE, EMB), lambda i: (i, 0)),
            pl.BlockSpec((2, EMB), lambda i: (0, 0)),
            pl.BlockSpec((EMB, EMB), lambda i: (0, 0)),
            pl.BlockSpec((1, EMB), lambda i: (0, 0)),
            pl.BlockSpec((1, EMB), lambda i: (0, 0)),
            pl.BlockSpec((1, EMB), lambda i: (0, 0)),
        ],
        out_specs=[
            pl.BlockSpec((BE, EMB), lambda i: (i, 0)),
            pl.BlockSpec((2, EMB), lambda i: (0, 0)),
        ],
        out_shape=[
            jax.ShapeDtypeStruct((E_PAD, EMB), jnp.float32),
            jax.ShapeDtypeStruct((2, EMB), jnp.float32),
        ],
        scratch_shapes=[pltpu.VMEM((8, EMB), jnp.float32)],
    )(y1, ss1, w2, b2, g2, be2)


def _bn_tc(y, g, be):
    mu = jnp.mean(y, axis=0)
    var = jnp.mean((y - mu) ** 2, axis=0)
    return g * (y - mu) * lax.rsqrt(var + EPS) + be


def _node_mid_body(h_ref, ap_ref, wuh_ref, wua_ref, bu1_ref, gu1_ref, beu1_ref,
                   wu2_ref, bu2_ref, gu2_ref, beu2_ref, h_out):
    h = h_ref[...]
    aggr = ap_ref[0] + ap_ref[1]
    y = (jnp.dot(h, wuh_ref[...], preferred_element_type=jnp.float32)
         + jnp.dot(aggr, wua_ref[...], preferred_element_type=jnp.float32)
         + bu1_ref[...])
    z = jnp.maximum(_bn_tc(y, gu1_ref[...], beu1_ref[...]), 0.0)
    y2 = jnp.dot(z, wu2_ref[...], preferred_element_type=jnp.float32) + bu2_ref[...]
    u = jnp.maximum(_bn_tc(y2, gu2_ref[...], beu2_ref[...]), 0.0)
    h_out[...] = h + u


def _tc_node_mid(h, ap, wuh, wua, bu1, gu1, beu1, wu2, bu2, gu2, beu2):
    return pl.pallas_call(
        _node_mid_body,
        out_shape=jax.ShapeDtypeStruct((N, EMB), jnp.float32),
    )(h, ap, wuh, wua, bu1, gu1, beu1, wu2, bu2, gu2, beu2)


def _node_last_body(h_ref, ap_ref, wuh_ref, wua_ref, bu1_ref, gu1_ref, beu1_ref,
                    wu2_ref, bu2_ref, gu2_ref, beu2_ref, wp_ref, bp_ref,
                    out_ref):
    h = h_ref[...]
    aggr = ap_ref[0] + ap_ref[1]
    y = (jnp.dot(h, wuh_ref[...], preferred_element_type=jnp.float32)
         + jnp.dot(aggr, wua_ref[...], preferred_element_type=jnp.float32)
         + bu1_ref[...])
    z = jnp.maximum(_bn_tc(y, gu1_ref[...], beu1_ref[...]), 0.0)
    y2 = jnp.dot(z, wu2_ref[...], preferred_element_type=jnp.float32) + bu2_ref[...]
    u = jnp.maximum(_bn_tc(y2, gu2_ref[...], beu2_ref[...]), 0.0)
    hn = h + u
    out_ref[...] = jnp.sum(hn * wp_ref[...], axis=1, keepdims=True) + bp_ref[...]


def _tc_node_last(h, ap, wuh, wua, bu1, gu1, beu1, wu2, bu2, gu2, beu2, wp, bp):
    return pl.pallas_call(
        _node_last_body,
        out_shape=jax.ShapeDtypeStruct((N, 1), jnp.float32),
    )(h, ap, wuh, wua, bu1, gu1, beu1, wu2, bu2, gu2, beu2, wp, bp)


def kernel(x, edge_index, edge_attr, params):
    src = edge_index[0]
    dst = edge_index[1]
    pad = E_PAD - E
    dst3 = jnp.pad(dst, (0, pad)).reshape(NW, NBPW_G, KG)
    src3 = jnp.pad(src, (0, pad)).reshape(NW, NBPW_G, KG)
    dsts3 = jnp.pad(dst, (0, pad), constant_values=PAD_SINK).reshape(NW, NBPW, KB)

    row = lambda v: v.reshape(1, EMB)
    lp = params['layers']
    wi = params['lin_in']['W']
    bi = row(params['lin_in']['b'])

    wd = [l['msg']['W1'][:EMB] for l in lp]
    ws = [l['msg']['W1'][EMB:2 * EMB] for l in lp]
    wc = [l['msg']['W1'][2 * EMB:] for l in lp]

    h = _tc_init(x, wi, bi)
    hd, hs = _tc_tables(h, wd[0], ws[0])

    out = None
    for l in range(4):
        m = lp[l]['msg']
        u = lp[l]['upd']
        g = _sc_gather(hd, hs, dst3, src3)
        y1, ss1 = _tc_msg1(g, edge_attr, wc[l], row(m['b1']),
                           row(m['g1']), row(m['be1']))
        y2, ss2 = _tc_msg2(y1, ss1, m['W2'], row(m['b2']),
                           row(m['g2']), row(m['be2']))
        ap = _sc_scatter(y2, dsts3, ss2)
        uargs = (u['W1'][:EMB], u['W1'][EMB:], row(u['b1']),
                 row(u['g1']), row(u['be1']),
                 u['W2'], row(u['b2']), row(u['g2']), row(u['be2']))
        if l < 3:
            h = _tc_node_mid(h, ap, *uargs)
            hd, hs = _tc_tables(h, wd[l + 1], ws[l + 1])
        else:
            wp = params['lin_pred']['W'].reshape(1, EMB)
            bp = params['lin_pred']['b'].reshape(1, 1)
            out = _tc_node_last(h, ap, *uargs, wp, bp)
    return out.reshape(-1)
